# baseline (speedup 1.0000x reference)
import functools

import jax
import jax.numpy as jnp
from jax import lax
from jax.experimental import pallas as pl
from jax.experimental.pallas import tpu as pltpu
from jax.experimental.pallas import tpu_sc as plsc

ESIZE = 64
NBUF = 8

_info = plsc.get_sparse_core_info()
NC, NS = _info.num_cores, _info.num_subcores
NW = NC * NS


@jax.jit
def _gather_rows(x, emb):
    nb, nseq = x.shape
    assert nseq <= 128
    xrows_per_w = nb // NW
    ngrp = xrows_per_w // NBUF
    assert ngrp * NBUF * NW == nb
    mesh = plsc.VectorSubcoreMesh(core_axis_name="c", subcore_axis_name="s")

    @functools.partial(
        pl.kernel,
        out_type=jax.ShapeDtypeStruct((nb, nseq, ESIZE), jnp.float32),
        mesh=mesh,
        scratch_types=[
            pltpu.VMEM((xrows_per_w, nseq), jnp.int32),
            pltpu.VMEM((NBUF, nseq, ESIZE), jnp.float32),
            pltpu.SemaphoreType.DMA((NBUF,)),
            pltpu.SemaphoreType.DMA((NBUF,)),
        ],
        compiler_params=pltpu.CompilerParams(use_tc_tiling_on_sc=False),
    )
    def k(emb_hbm, x_hbm, out_hbm, idx_v, rows_v, gsem, ssem):
        wid = lax.axis_index("s") * NC + lax.axis_index("c")
        base = wid * xrows_per_w
        pltpu.sync_copy(x_hbm.at[pl.ds(base, xrows_per_w)], idx_v)

        def gather(r, b):
            return pltpu.make_async_copy(
                emb_hbm.at[idx_v.at[r]], rows_v.at[b], gsem.at[b]
            )

        def store(r, b):
            return pltpu.make_async_copy(
                rows_v.at[b], out_hbm.at[base + r], ssem.at[b]
            )

        for b in range(NBUF):
            gather(b, b).start()

        def group(g, _):
            r0 = g * NBUF
            for b in range(NBUF):
                r = r0 + b
                gather(r, b).wait()
                store(r, b).start()
                store(r, b).wait()
                gather(r + NBUF, b).start()
            return _

        lax.fori_loop(0, ngrp - 1, group, None)

        r0 = (ngrp - 1) * NBUF
        for b in range(NBUF):
            r = r0 + b
            gather(r, b).wait()
            store(r, b).start()
            store(r, b).wait()

    return k(emb, x)


def kernel(x, emb):
    return _gather_rows(x.astype(jnp.int32), emb)
